# initial kernel scaffold (unmeasured)
import jax
import jax.numpy as jnp
from jax import lax
from jax.experimental import pallas as pl
from jax.experimental.pallas import tpu as pltpu

N_DEV = 16


def kernel(t, W):
    m, k = t.shape
    _, n = W.shape
    chunk = m // N_DEV

    def body(t_ref, w_ref, out_ref, send_buf, recv_buf, send_sems,
             rs_recv_sems, ag_recv_sems):
        d = lax.axis_index("i")
        right = lax.rem(d + 1, N_DEV)
        left = lax.rem(d + N_DEV - 1, N_DEV)

        barrier_sem = pltpu.get_barrier_semaphore()
        for nbr in (left, right):
            pl.semaphore_signal(
                barrier_sem, inc=1,
                device_id=(nbr,), device_id_type=pl.DeviceIdType.MESH,
            )
        pl.semaphore_wait(barrier_sem, 2)

        def t_chunk(idx):
            return t_ref[pl.ds(idx * chunk, chunk), :]

        send_buf[0, :, :] = t_chunk(d)
        for s in range(N_DEV - 1):
            slot = s % 2
            rdma = pltpu.make_async_remote_copy(
                src_ref=send_buf.at[slot],
                dst_ref=recv_buf.at[s],
                send_sem=send_sems.at[slot],
                recv_sem=rs_recv_sems.at[s],
                device_id=(right,),
                device_id_type=pl.DeviceIdType.MESH,
            )
            rdma.start()
            rdma.wait()
            got = lax.rem(d + N_DEV - 1 - s, N_DEV)
            if s < N_DEV - 2:
                send_buf[(s + 1) % 2, :, :] = recv_buf[s] + t_chunk(got)
            else:
                red = recv_buf[s] + t_chunk(got)
                out_ref[pl.ds(got * chunk, chunk), :] = jnp.dot(
                    red, w_ref[:, :], preferred_element_type=jnp.float32
                )

        for h in range(N_DEV - 1):
            c = lax.rem(d + 1 - h + N_DEV, N_DEV)
            rdma = pltpu.make_async_remote_copy(
                src_ref=out_ref.at[pl.ds(c * chunk, chunk), :],
                dst_ref=out_ref.at[pl.ds(c * chunk, chunk), :],
                send_sem=send_sems.at[h % 2],
                recv_sem=ag_recv_sems.at[h],
                device_id=(right,),
                device_id_type=pl.DeviceIdType.MESH,
            )
            rdma.start()
            rdma.wait()

    return pl.pallas_call(
        body,
        out_shape=jax.ShapeDtypeStruct((m, n), jnp.float32),
        in_specs=[
            pl.BlockSpec(memory_space=pltpu.VMEM),
            pl.BlockSpec(memory_space=pltpu.VMEM),
        ],
        out_specs=pl.BlockSpec(memory_space=pltpu.VMEM),
        scratch_shapes=[
            pltpu.VMEM((2, chunk, k), jnp.float32),
            pltpu.VMEM((N_DEV - 1, chunk, k), jnp.float32),
            pltpu.SemaphoreType.DMA((2,)),
            pltpu.SemaphoreType.DMA((N_DEV - 1,)),
            pltpu.SemaphoreType.DMA((N_DEV - 1,)),
        ],
        compiler_params=pltpu.CompilerParams(collective_id=0),
    )(t, W)


# baseline (device time: 423743 ns/iter reference)
import jax
import jax.numpy as jnp
from jax import lax
from jax.experimental import pallas as pl
from jax.experimental.pallas import tpu as pltpu

N_DEV = 16


def kernel(t, W):
    m, k = t.shape
    _, n = W.shape
    chunk = m // N_DEV

    def body(t_ref, w_ref, out_ref, send_buf, recv_buf, send_sems,
             rs_recv_sems, ag_recv_sems):
        d = lax.axis_index("i")
        right = lax.rem(d + 1, N_DEV)
        left = lax.rem(d + N_DEV - 1, N_DEV)

        barrier_sem = pltpu.get_barrier_semaphore()
        for nbr in (left, right):
            pl.semaphore_signal(
                barrier_sem, inc=1,
                device_id=(nbr,), device_id_type=pl.DeviceIdType.MESH,
            )
        pl.semaphore_wait(barrier_sem, 2)

        def t_chunk(idx):
            return t_ref[pl.ds(idx * chunk, chunk), :]

        send_buf[0, :, :] = t_chunk(d)
        for s in range(N_DEV - 1):
            slot = s % 2
            rdma = pltpu.make_async_remote_copy(
                src_ref=send_buf.at[slot],
                dst_ref=recv_buf.at[s],
                send_sem=send_sems.at[slot],
                recv_sem=rs_recv_sems.at[s],
                device_id=(right,),
                device_id_type=pl.DeviceIdType.MESH,
            )
            rdma.start()
            rdma.wait()
            got = lax.rem(d + N_DEV - 1 - s, N_DEV)
            if s < N_DEV - 2:
                send_buf[(s + 1) % 2, :, :] = recv_buf[s] + t_chunk(got)
            else:
                red = recv_buf[s] + t_chunk(got)
                out_ref[pl.ds(got * chunk, chunk), :] = jnp.dot(
                    red, w_ref[:, :], preferred_element_type=jnp.float32
                )

        for h in range(N_DEV - 1):
            c = lax.rem(d + 1 - h + N_DEV, N_DEV)
            rdma = pltpu.make_async_remote_copy(
                src_ref=out_ref.at[pl.ds(c * chunk, chunk), :],
                dst_ref=out_ref.at[pl.ds(c * chunk, chunk), :],
                send_sem=send_sems.at[h % 2],
                recv_sem=ag_recv_sems.at[h],
                device_id=(right,),
                device_id_type=pl.DeviceIdType.MESH,
            )
            rdma.start()
            rdma.wait()

    return pl.pallas_call(
        body,
        out_shape=jax.ShapeDtypeStruct((m, n), jnp.float32),
        in_specs=[
            pl.BlockSpec(memory_space=pltpu.VMEM),
            pl.BlockSpec(memory_space=pltpu.VMEM),
        ],
        out_specs=pl.BlockSpec(memory_space=pltpu.VMEM),
        scratch_shapes=[
            pltpu.VMEM((2, chunk, k), jnp.float32),
            pltpu.VMEM((N_DEV - 1, chunk, k), jnp.float32),
            pltpu.SemaphoreType.DMA((2,)),
            pltpu.SemaphoreType.DMA((N_DEV - 1,)),
            pltpu.SemaphoreType.DMA((N_DEV - 1,)),
        ],
        compiler_params=pltpu.CompilerParams(
            collective_id=0, vmem_limit_bytes=100 * 1024 * 1024
        ),
    )(t, W)


# device time: 294677 ns/iter; 1.4380x vs baseline; 1.4380x over previous
import jax
import jax.numpy as jnp
from jax import lax
from jax.experimental import pallas as pl
from jax.experimental.pallas import tpu as pltpu

N_DEV = 16


def kernel(t, W):
    m, k = t.shape
    _, n = W.shape
    chunk = m // N_DEV
    half = chunk // 2

    def body(t_ref, w_ref, out_ref, send_cw, send_ccw, recv_cw, recv_ccw,
             send_sems_cw, send_sems_ccw, rs_sems_cw, rs_sems_ccw,
             ag_sems_cw, ag_sems_ccw):
        d = lax.axis_index("i")
        right = lax.rem(d + 1, N_DEV)
        left = lax.rem(d + N_DEV - 1, N_DEV)

        barrier_sem = pltpu.get_barrier_semaphore()
        for nbr in (left, right):
            pl.semaphore_signal(
                barrier_sem, inc=1,
                device_id=(nbr,), device_id_type=pl.DeviceIdType.MESH,
            )
        pl.semaphore_wait(barrier_sem, 2)

        def tA(idx):
            return t_ref[pl.ds(idx * chunk, half), :]

        def tB(idx):
            return t_ref[pl.ds(idx * chunk + half, half), :]

        def hop(src, dst, s_sem, r_sem, dev):
            return pltpu.make_async_remote_copy(
                src_ref=src, dst_ref=dst, send_sem=s_sem, recv_sem=r_sem,
                device_id=(dev,), device_id_type=pl.DeviceIdType.MESH,
            )

        send_cw[0, :, :] = tA(d)
        send_ccw[0, :, :] = tB(d)
        for s in range(N_DEV - 1):
            slot = s % 2
            r1 = hop(send_cw.at[slot], recv_cw.at[s],
                     send_sems_cw.at[slot], rs_sems_cw.at[s], right)
            r2 = hop(send_ccw.at[slot], recv_ccw.at[s],
                     send_sems_ccw.at[slot], rs_sems_ccw.at[s], left)
            r1.start()
            r2.start()
            r1.wait()
            r2.wait()
            got_cw = lax.rem(d + N_DEV - 1 - s, N_DEV)
            got_ccw = lax.rem(d + 1 + s, N_DEV)
            if s < N_DEV - 2:
                send_cw[(s + 1) % 2, :, :] = recv_cw[s] + tA(got_cw)
                send_ccw[(s + 1) % 2, :, :] = recv_ccw[s] + tB(got_ccw)
            else:
                red_a = recv_cw[s] + tA(got_cw)
                red_b = recv_ccw[s] + tB(got_ccw)
                out_ref[pl.ds(got_cw * chunk, half), :] = jnp.dot(
                    red_a, w_ref[:, :], preferred_element_type=jnp.float32
                )
                out_ref[pl.ds(got_ccw * chunk + half, half), :] = jnp.dot(
                    red_b, w_ref[:, :], preferred_element_type=jnp.float32
                )

        for h in range(N_DEV - 1):
            ca = lax.rem(d + 1 - h + N_DEV, N_DEV)
            cb = lax.rem(d - 1 + h + N_DEV, N_DEV)
            rowsA = pl.ds(ca * chunk, half)
            rowsB = pl.ds(cb * chunk + half, half)
            r1 = hop(out_ref.at[rowsA, :], out_ref.at[rowsA, :],
                     send_sems_cw.at[h % 2], ag_sems_cw.at[h], right)
            r2 = hop(out_ref.at[rowsB, :], out_ref.at[rowsB, :],
                     send_sems_ccw.at[h % 2], ag_sems_ccw.at[h], left)
            r1.start()
            r2.start()
            r1.wait()
            r2.wait()

    return pl.pallas_call(
        body,
        out_shape=jax.ShapeDtypeStruct((m, n), jnp.float32),
        in_specs=[
            pl.BlockSpec(memory_space=pltpu.VMEM),
            pl.BlockSpec(memory_space=pltpu.VMEM),
        ],
        out_specs=pl.BlockSpec(memory_space=pltpu.VMEM),
        scratch_shapes=[
            pltpu.VMEM((2, half, k), jnp.float32),
            pltpu.VMEM((2, half, k), jnp.float32),
            pltpu.VMEM((N_DEV - 1, half, k), jnp.float32),
            pltpu.VMEM((N_DEV - 1, half, k), jnp.float32),
            pltpu.SemaphoreType.DMA((2,)),
            pltpu.SemaphoreType.DMA((2,)),
            pltpu.SemaphoreType.DMA((N_DEV - 1,)),
            pltpu.SemaphoreType.DMA((N_DEV - 1,)),
            pltpu.SemaphoreType.DMA((N_DEV - 1,)),
            pltpu.SemaphoreType.DMA((N_DEV - 1,)),
        ],
        compiler_params=pltpu.CompilerParams(
            collective_id=0, vmem_limit_bytes=100 * 1024 * 1024
        ),
    )(t, W)


# device time: 215331 ns/iter; 1.9679x vs baseline; 1.3685x over previous
import jax
import jax.numpy as jnp
from jax import lax
from jax.experimental import pallas as pl
from jax.experimental.pallas import tpu as pltpu

N_DEV = 16
N_SUB = 4


def kernel(t, W):
    m, k = t.shape
    _, n = W.shape
    chunk = m // N_DEV
    sub = chunk // N_SUB

    def body(t_ref, w_ref, out_ref, send_buf, recv_buf, send_sems,
             rs_sems, ag_sems):
        d = lax.axis_index("i")
        right = lax.rem(d + 1, N_DEV)
        left = lax.rem(d + N_DEV - 1, N_DEV)

        streams = [(0, True), (sub, True), (2 * sub, False), (3 * sub, False)]

        barrier_sem = pltpu.get_barrier_semaphore()
        for nbr in (left, right):
            pl.semaphore_signal(
                barrier_sem, inc=1,
                device_id=(nbr,), device_id_type=pl.DeviceIdType.MESH,
            )
        pl.semaphore_wait(barrier_sem, 2)

        def tsub(idx, off):
            return t_ref[pl.ds(idx * chunk + off, sub), :]

        def rs_desc(i, s, slot, dev):
            return pltpu.make_async_remote_copy(
                src_ref=send_buf.at[i, slot],
                dst_ref=recv_buf.at[i, s],
                send_sem=send_sems.at[i, slot],
                recv_sem=rs_sems.at[i, s],
                device_id=(dev,), device_id_type=pl.DeviceIdType.MESH,
            )

        def ag_desc(i, h, c, off, dev):
            rows = pl.ds(c * chunk + off, sub)
            return pltpu.make_async_remote_copy(
                src_ref=out_ref.at[rows, :],
                dst_ref=out_ref.at[rows, :],
                send_sem=send_sems.at[i, h % 2],
                recv_sem=ag_sems.at[i, h],
                device_id=(dev,), device_id_type=pl.DeviceIdType.MESH,
            )

        rs_sent = [[] for _ in streams]
        for i, (off, cw) in enumerate(streams):
            send_buf[i, 0, :, :] = tsub(d, off)
            dsc = rs_desc(i, 0, 0, right if cw else left)
            dsc.start()
            rs_sent[i].append(dsc)

        for s in range(N_DEV - 1):
            for i, (off, cw) in enumerate(streams):
                dev = right if cw else left
                rs_desc(i, s, 0, dev).wait_recv()
                if cw:
                    got = lax.rem(d + N_DEV - 1 - s, N_DEV)
                else:
                    got = lax.rem(d + 1 + s, N_DEV)
                if s < N_DEV - 2:
                    slot = (s + 1) % 2
                    if s >= 1:
                        rs_sent[i][s - 1].wait_send()
                    send_buf[i, slot, :, :] = recv_buf[i, s] + tsub(got, off)
                    dsc = rs_desc(i, s + 1, slot, dev)
                    dsc.start()
                    rs_sent[i].append(dsc)
                else:
                    red = recv_buf[i, s] + tsub(got, off)
                    out_ref[pl.ds(got * chunk + off, sub), :] = jnp.dot(
                        red, w_ref[:, :], preferred_element_type=jnp.float32
                    )
        for i in range(len(streams)):
            rs_sent[i][N_DEV - 3].wait_send()
            rs_sent[i][N_DEV - 2].wait_send()

        def ag_chunk(h, cw):
            if cw:
                return lax.rem(d + 1 - h + N_DEV + N_DEV, N_DEV)
            return lax.rem(d - 1 + h + N_DEV, N_DEV)

        ag_sent = [[] for _ in streams]
        for i, (off, cw) in enumerate(streams):
            dsc = ag_desc(i, 0, ag_chunk(0, cw), off, right if cw else left)
            dsc.start()
            ag_sent[i].append(dsc)

        for h in range(N_DEV - 1):
            for i, (off, cw) in enumerate(streams):
                dev = right if cw else left
                ag_desc(i, h, ag_chunk(h, cw), off, dev).wait_recv()
                if h < N_DEV - 2:
                    if h >= 1:
                        ag_sent[i][h - 1].wait_send()
                    dsc = ag_desc(i, h + 1, ag_chunk(h + 1, cw), off, dev)
                    dsc.start()
                    ag_sent[i].append(dsc)
        for i in range(len(streams)):
            ag_sent[i][N_DEV - 3].wait_send()
            ag_sent[i][N_DEV - 2].wait_send()

    return pl.pallas_call(
        body,
        out_shape=jax.ShapeDtypeStruct((m, n), jnp.float32),
        in_specs=[
            pl.BlockSpec(memory_space=pltpu.VMEM),
            pl.BlockSpec(memory_space=pltpu.VMEM),
        ],
        out_specs=pl.BlockSpec(memory_space=pltpu.VMEM),
        scratch_shapes=[
            pltpu.VMEM((N_SUB, 2, sub, k), jnp.float32),
            pltpu.VMEM((N_SUB, N_DEV - 1, sub, k), jnp.float32),
            pltpu.SemaphoreType.DMA((N_SUB, 2)),
            pltpu.SemaphoreType.DMA((N_SUB, N_DEV - 1)),
            pltpu.SemaphoreType.DMA((N_SUB, N_DEV - 1)),
        ],
        compiler_params=pltpu.CompilerParams(
            collective_id=0, vmem_limit_bytes=100 * 1024 * 1024
        ),
    )(t, W)


# device time: 155156 ns/iter; 2.7311x vs baseline; 1.3878x over previous
import jax
import jax.numpy as jnp
from jax import lax
from jax.experimental import pallas as pl
from jax.experimental.pallas import tpu as pltpu

N_DEV = 16
N_SUB = 4


def kernel(t, W):
    m, k = t.shape
    _, n = W.shape
    chunk = m // N_DEV
    sub = chunk // N_SUB

    def body(t_ref, w_ref, out_ref, send_buf, recv_buf, ag_buf, send_sems,
             rs_sems, ag_sems):
        d = lax.axis_index("i")
        right = lax.rem(d + 1, N_DEV)
        left = lax.rem(d + N_DEV - 1, N_DEV)

        streams = [(0, True), (sub, True), (2 * sub, False), (3 * sub, False)]

        barrier_sem = pltpu.get_barrier_semaphore()
        for nbr in (left, right):
            pl.semaphore_signal(
                barrier_sem, inc=1,
                device_id=(nbr,), device_id_type=pl.DeviceIdType.MESH,
            )
        pl.semaphore_wait(barrier_sem, 2)

        def tsub(idx, off):
            return t_ref[pl.ds(idx * chunk + off, sub), :]

        def rs_desc(i, s, slot, dev):
            return pltpu.make_async_remote_copy(
                src_ref=send_buf.at[i, slot],
                dst_ref=recv_buf.at[i, s],
                send_sem=send_sems.at[i, slot],
                recv_sem=rs_sems.at[i, s],
                device_id=(dev,), device_id_type=pl.DeviceIdType.MESH,
            )

        def ag_desc(i, h, src_slot, dev):
            return pltpu.make_async_remote_copy(
                src_ref=ag_buf.at[i, src_slot],
                dst_ref=ag_buf.at[i, h],
                send_sem=send_sems.at[i, h % 2],
                recv_sem=ag_sems.at[i, h],
                device_id=(dev,), device_id_type=pl.DeviceIdType.MESH,
            )

        rs_sent = [[] for _ in streams]
        for i, (off, cw) in enumerate(streams):
            send_buf[i, 0, :, :] = tsub(d, off).astype(jnp.bfloat16)
            dsc = rs_desc(i, 0, 0, right if cw else left)
            dsc.start()
            rs_sent[i].append(dsc)

        for s in range(N_DEV - 1):
            for i, (off, cw) in enumerate(streams):
                dev = right if cw else left
                rs_desc(i, s, 0, dev).wait_recv()
                if cw:
                    got = lax.rem(d + N_DEV - 1 - s, N_DEV)
                else:
                    got = lax.rem(d + 1 + s, N_DEV)
                acc = recv_buf[i, s].astype(jnp.float32) + tsub(got, off)
                if s < N_DEV - 2:
                    slot = (s + 1) % 2
                    if s >= 1:
                        rs_sent[i][s - 1].wait_send()
                    send_buf[i, slot, :, :] = acc.astype(jnp.bfloat16)
                    dsc = rs_desc(i, s + 1, slot, dev)
                    dsc.start()
                    rs_sent[i].append(dsc)
                else:
                    res = jnp.dot(
                        acc, w_ref[:, :], preferred_element_type=jnp.float32
                    )
                    out_ref[pl.ds(got * chunk + off, sub), :] = res
                    ag_buf[i, N_DEV - 1, :, :] = res.astype(jnp.bfloat16)
        for i in range(len(streams)):
            rs_sent[i][N_DEV - 3].wait_send()
            rs_sent[i][N_DEV - 2].wait_send()

        def ag_recv_chunk(h, cw):
            if cw:
                return lax.rem(d - h + 2 * N_DEV, N_DEV)
            return lax.rem(d + h, N_DEV)

        ag_sent = [[] for _ in streams]
        for i, (off, cw) in enumerate(streams):
            dsc = ag_desc(i, 0, N_DEV - 1, right if cw else left)
            dsc.start()
            ag_sent[i].append(dsc)

        for h in range(N_DEV - 1):
            for i, (off, cw) in enumerate(streams):
                dev = right if cw else left
                rc = ag_recv_chunk(h, cw)
                ag_desc(i, h, h, dev).wait_recv()
                out_ref[pl.ds(rc * chunk + off, sub), :] = (
                    ag_buf[i, h].astype(jnp.float32)
                )
                if h < N_DEV - 2:
                    if h >= 1:
                        ag_sent[i][h - 1].wait_send()
                    dsc = ag_desc(i, h + 1, h, dev)
                    dsc.start()
                    ag_sent[i].append(dsc)
        for i in range(len(streams)):
            ag_sent[i][N_DEV - 3].wait_send()
            ag_sent[i][N_DEV - 2].wait_send()

    return pl.pallas_call(
        body,
        out_shape=jax.ShapeDtypeStruct((m, n), jnp.float32),
        in_specs=[
            pl.BlockSpec(memory_space=pltpu.VMEM),
            pl.BlockSpec(memory_space=pltpu.VMEM),
        ],
        out_specs=pl.BlockSpec(memory_space=pltpu.VMEM),
        scratch_shapes=[
            pltpu.VMEM((N_SUB, 2, sub, k), jnp.bfloat16),
            pltpu.VMEM((N_SUB, N_DEV - 1, sub, k), jnp.bfloat16),
            pltpu.VMEM((N_SUB, N_DEV, sub, k), jnp.bfloat16),
            pltpu.SemaphoreType.DMA((N_SUB, 2)),
            pltpu.SemaphoreType.DMA((N_SUB, N_DEV - 1)),
            pltpu.SemaphoreType.DMA((N_SUB, N_DEV - 1)),
        ],
        compiler_params=pltpu.CompilerParams(
            collective_id=0, vmem_limit_bytes=100 * 1024 * 1024
        ),
    )(t, W)


# device time: 151816 ns/iter; 2.7912x vs baseline; 1.0220x over previous
import jax
import jax.numpy as jnp
from jax import lax
from jax.experimental import pallas as pl
from jax.experimental.pallas import tpu as pltpu

N_DEV = 16
N_SUB = 4


def kernel(t, W):
    m, k = t.shape
    _, n = W.shape
    chunk = m // N_DEV
    sub = chunk // N_SUB

    def body(t_ref, w_ref, out_ref, send_buf, recv_buf, ag_buf, send_sems,
             rs_sems, ag_sems):
        d = lax.axis_index("i")
        right = lax.rem(d + 1, N_DEV)
        left = lax.rem(d + N_DEV - 1, N_DEV)

        streams = [(0, True), (sub, True), (2 * sub, False), (3 * sub, False)]

        barrier_sem = pltpu.get_barrier_semaphore()
        for nbr in (left, right):
            pl.semaphore_signal(
                barrier_sem, inc=1,
                device_id=(nbr,), device_id_type=pl.DeviceIdType.MESH,
            )
        pl.semaphore_wait(barrier_sem, 2)

        def tsub(idx, off):
            return t_ref[pl.ds(idx * chunk + off, sub), :]

        def rs_desc(i, s, slot, dev):
            return pltpu.make_async_remote_copy(
                src_ref=send_buf.at[i, slot],
                dst_ref=recv_buf.at[i, s],
                send_sem=send_sems.at[i, slot],
                recv_sem=rs_sems.at[i, s],
                device_id=(dev,), device_id_type=pl.DeviceIdType.MESH,
            )

        def ag_desc(i, h, src_slot, dev):
            return pltpu.make_async_remote_copy(
                src_ref=ag_buf.at[i, src_slot],
                dst_ref=ag_buf.at[i, h],
                send_sem=send_sems.at[i, h % 2],
                recv_sem=ag_sems.at[i, h],
                device_id=(dev,), device_id_type=pl.DeviceIdType.MESH,
            )

        rs_sent = [[] for _ in streams]
        ag_sent = [[] for _ in streams]
        for i, (off, cw) in enumerate(streams):
            send_buf[i, 0, :, :] = tsub(d, off).astype(jnp.bfloat16)
            dsc = rs_desc(i, 0, 0, right if cw else left)
            dsc.start()
            rs_sent[i].append(dsc)

        for s in range(N_DEV - 1):
            for i, (off, cw) in enumerate(streams):
                dev = right if cw else left
                rs_desc(i, s, 0, dev).wait_recv()
                if cw:
                    got = lax.rem(d + N_DEV - 1 - s, N_DEV)
                else:
                    got = lax.rem(d + 1 + s, N_DEV)
                acc = recv_buf[i, s].astype(jnp.float32) + tsub(got, off)
                if s < N_DEV - 2:
                    slot = (s + 1) % 2
                    if s >= 1:
                        rs_sent[i][s - 1].wait_send()
                    send_buf[i, slot, :, :] = acc.astype(jnp.bfloat16)
                    dsc = rs_desc(i, s + 1, slot, dev)
                    dsc.start()
                    rs_sent[i].append(dsc)
                else:
                    res = jnp.dot(
                        acc, w_ref[:, :], preferred_element_type=jnp.float32
                    )
                    ag_buf[i, N_DEV - 1, :, :] = res.astype(jnp.bfloat16)
                    rs_sent[i][N_DEV - 3].wait_send()
                    rs_sent[i][N_DEV - 2].wait_send()
                    dsc = ag_desc(i, 0, N_DEV - 1, dev)
                    dsc.start()
                    ag_sent[i].append(dsc)
                    out_ref[pl.ds(got * chunk + off, sub), :] = res

        def ag_recv_chunk(h, cw):
            if cw:
                return lax.rem(d - h + 2 * N_DEV, N_DEV)
            return lax.rem(d + h, N_DEV)

        for h in range(N_DEV - 1):
            for i, (off, cw) in enumerate(streams):
                dev = right if cw else left
                rc = ag_recv_chunk(h, cw)
                ag_desc(i, h, h, dev).wait_recv()
                if h < N_DEV - 2:
                    if h >= 1:
                        ag_sent[i][h - 1].wait_send()
                    dsc = ag_desc(i, h + 1, h, dev)
                    dsc.start()
                    ag_sent[i].append(dsc)
                out_ref[pl.ds(rc * chunk + off, sub), :] = (
                    ag_buf[i, h].astype(jnp.float32)
                )
        for i in range(len(streams)):
            ag_sent[i][N_DEV - 3].wait_send()
            ag_sent[i][N_DEV - 2].wait_send()

    return pl.pallas_call(
        body,
        out_shape=jax.ShapeDtypeStruct((m, n), jnp.float32),
        in_specs=[
            pl.BlockSpec(memory_space=pltpu.VMEM),
            pl.BlockSpec(memory_space=pltpu.VMEM),
        ],
        out_specs=pl.BlockSpec(memory_space=pltpu.VMEM),
        scratch_shapes=[
            pltpu.VMEM((N_SUB, 2, sub, k), jnp.bfloat16),
            pltpu.VMEM((N_SUB, N_DEV - 1, sub, k), jnp.bfloat16),
            pltpu.VMEM((N_SUB, N_DEV, sub, k), jnp.bfloat16),
            pltpu.SemaphoreType.DMA((N_SUB, 2)),
            pltpu.SemaphoreType.DMA((N_SUB, N_DEV - 1)),
            pltpu.SemaphoreType.DMA((N_SUB, N_DEV - 1)),
        ],
        compiler_params=pltpu.CompilerParams(
            collective_id=0, vmem_limit_bytes=100 * 1024 * 1024
        ),
    )(t, W)


# device time: 127672 ns/iter; 3.3190x vs baseline; 1.1891x over previous
import jax
import jax.numpy as jnp
from jax import lax
from jax.experimental import pallas as pl
from jax.experimental.pallas import tpu as pltpu

N_DEV = 16
R_HOPS = 8
L_HOPS = 7


def kernel(t, W):
    m, k = t.shape
    _, n = W.shape
    chunk = m // N_DEV
    sub = chunk // 2

    def body(t_ref, w_ref, out_ref, send_buf, rs_recv_r, rs_recv_l,
             ag_own, ag_recv_r, ag_recv_l, send_sems,
             rs_sems_r, rs_sems_l, ag_sems_r, ag_sems_l):
        d = lax.axis_index("i")
        right = lax.rem(d + 1, N_DEV)
        left = lax.rem(d + N_DEV - 1, N_DEV)

        streams = [(0, True), (sub, True), (0, False), (sub, False)]
        n_hops = {True: R_HOPS, False: L_HOPS}

        barrier_sem = pltpu.get_barrier_semaphore()
        for nbr in (left, right):
            pl.semaphore_signal(
                barrier_sem, inc=1,
                device_id=(nbr,), device_id_type=pl.DeviceIdType.MESH,
            )
        pl.semaphore_wait(barrier_sem, 2)

        def trows(c, off):
            return t_ref[pl.ds(c * chunk + off, sub), :]

        def rs_send_chunk(s, rwd):
            if rwd:
                return lax.rem(d + R_HOPS - s, N_DEV)
            return lax.rem(d + N_DEV - L_HOPS + s, N_DEV)

        def rs_desc(i, s, slot, rwd):
            return pltpu.make_async_remote_copy(
                src_ref=send_buf.at[i, slot],
                dst_ref=(rs_recv_r if rwd else rs_recv_l).at[i % 2, s],
                send_sem=send_sems.at[i, slot],
                recv_sem=(rs_sems_r if rwd else rs_sems_l).at[i % 2, s],
                device_id=(right if rwd else left,),
                device_id_type=pl.DeviceIdType.MESH,
            )

        def ag_desc(i, h, rwd):
            buf = ag_recv_r if rwd else ag_recv_l
            src = ag_own.at[i % 2] if h == 0 else buf.at[i % 2, h - 1]
            return pltpu.make_async_remote_copy(
                src_ref=src,
                dst_ref=buf.at[i % 2, h],
                send_sem=send_sems.at[i, h % 2],
                recv_sem=(ag_sems_r if rwd else ag_sems_l).at[i % 2, h],
                device_id=(right if rwd else left,),
                device_id_type=pl.DeviceIdType.MESH,
            )

        rs_sent = [[] for _ in streams]
        ag_sent = [[] for _ in streams]
        for i, (off, rwd) in enumerate(streams):
            send_buf[i, 0, :, :] = (
                trows(rs_send_chunk(0, rwd), off).astype(jnp.bfloat16)
            )
            dsc = rs_desc(i, 0, 0, rwd)
            dsc.start()
            rs_sent[i].append(dsc)

        for s in range(R_HOPS):
            for i, (off, rwd) in enumerate(streams):
                hops = n_hops[rwd]
                if s >= hops:
                    continue
                recv_buf = rs_recv_r if rwd else rs_recv_l
                rs_desc(i, s, 0, rwd).wait_recv()
                if s < hops - 1:
                    slot = (s + 1) % 2
                    if s >= 1:
                        rs_sent[i][s - 1].wait_send()
                    acc = recv_buf[i % 2, s].astype(jnp.float32) + trows(
                        rs_send_chunk(s + 1, rwd), off
                    )
                    send_buf[i, slot, :, :] = acc.astype(jnp.bfloat16)
                    dsc = rs_desc(i, s + 1, slot, rwd)
                    dsc.start()
                    rs_sent[i].append(dsc)

        for j in range(2):
            off = j * sub
            red = (
                rs_recv_r[j, R_HOPS - 1].astype(jnp.float32)
                + rs_recv_l[j, L_HOPS - 1].astype(jnp.float32)
                + trows(d, off)
            )
            res = jnp.dot(red, w_ref[:, :], preferred_element_type=jnp.float32)
            ag_own[j, :, :] = res.astype(jnp.bfloat16)
            for i, (ioff, rwd) in enumerate(streams):
                if ioff != off:
                    continue
                rs_sent[i][n_hops[rwd] - 2].wait_send()
                rs_sent[i][n_hops[rwd] - 1].wait_send()
                dsc = ag_desc(i, 0, rwd)
                dsc.start()
                ag_sent[i].append(dsc)
            out_ref[pl.ds(d * chunk + off, sub), :] = res

        for h in range(R_HOPS):
            for i, (off, rwd) in enumerate(streams):
                hops = n_hops[rwd]
                if h >= hops:
                    continue
                if rwd:
                    rc = lax.rem(d + N_DEV - 1 - h, N_DEV)
                else:
                    rc = lax.rem(d + 1 + h, N_DEV)
                ag_desc(i, h, rwd).wait_recv()
                if h < hops - 1:
                    if h >= 1:
                        ag_sent[i][h - 1].wait_send()
                    dsc = ag_desc(i, h + 1, rwd)
                    dsc.start()
                    ag_sent[i].append(dsc)
                buf = ag_recv_r if rwd else ag_recv_l
                out_ref[pl.ds(rc * chunk + off, sub), :] = (
                    buf[i % 2, h].astype(jnp.float32)
                )
        for i, (off, rwd) in enumerate(streams):
            hops = n_hops[rwd]
            ag_sent[i][hops - 2].wait_send()
            ag_sent[i][hops - 1].wait_send()

    return pl.pallas_call(
        body,
        out_shape=jax.ShapeDtypeStruct((m, n), jnp.float32),
        in_specs=[
            pl.BlockSpec(memory_space=pltpu.VMEM),
            pl.BlockSpec(memory_space=pltpu.VMEM),
        ],
        out_specs=pl.BlockSpec(memory_space=pltpu.VMEM),
        scratch_shapes=[
            pltpu.VMEM((4, 2, sub, k), jnp.bfloat16),
            pltpu.VMEM((2, R_HOPS, sub, k), jnp.bfloat16),
            pltpu.VMEM((2, L_HOPS, sub, k), jnp.bfloat16),
            pltpu.VMEM((2, sub, k), jnp.bfloat16),
            pltpu.VMEM((2, R_HOPS, sub, k), jnp.bfloat16),
            pltpu.VMEM((2, L_HOPS, sub, k), jnp.bfloat16),
            pltpu.SemaphoreType.DMA((4, 2)),
            pltpu.SemaphoreType.DMA((2, R_HOPS)),
            pltpu.SemaphoreType.DMA((2, L_HOPS)),
            pltpu.SemaphoreType.DMA((2, R_HOPS)),
            pltpu.SemaphoreType.DMA((2, L_HOPS)),
        ],
        compiler_params=pltpu.CompilerParams(
            collective_id=0, vmem_limit_bytes=100 * 1024 * 1024
        ),
    )(t, W)


# device time: 127639 ns/iter; 3.3199x vs baseline; 1.0003x over previous
import jax
import jax.numpy as jnp
from jax import lax
from jax.experimental import pallas as pl
from jax.experimental.pallas import tpu as pltpu

N_DEV = 16
R_HOPS = 8
L_HOPS = 7


def kernel(t, W):
    m, k = t.shape
    _, n = W.shape
    chunk = m // N_DEV
    sub = chunk // 2

    def body(t_ref, w_ref, out_ref, send_buf, rs_recv_r, rs_recv_l,
             ag_own, ag_recv_r, ag_recv_l, send_sems,
             rs_sems_r, rs_sems_l, ag_sems_r, ag_sems_l):
        d = lax.axis_index("i")
        right = lax.rem(d + 1, N_DEV)
        left = lax.rem(d + N_DEV - 1, N_DEV)

        streams = [(0, True), (sub, True), (0, False), (sub, False)]
        n_hops = {True: R_HOPS, False: L_HOPS}

        barrier_sem = pltpu.get_barrier_semaphore()
        for nbr in (left, right):
            pl.semaphore_signal(
                barrier_sem, inc=1,
                device_id=(nbr,), device_id_type=pl.DeviceIdType.MESH,
            )
        pl.semaphore_wait(barrier_sem, 2)

        def trows(c, off):
            return t_ref[pl.ds(c * chunk + off, sub), :]

        def rs_send_chunk(s, rwd):
            if rwd:
                return lax.rem(d + R_HOPS - s, N_DEV)
            return lax.rem(d + N_DEV - L_HOPS + s, N_DEV)

        def rs_desc(i, s, slot, rwd):
            return pltpu.make_async_remote_copy(
                src_ref=send_buf.at[i, slot],
                dst_ref=(rs_recv_r if rwd else rs_recv_l).at[i % 2, s],
                send_sem=send_sems.at[i, slot],
                recv_sem=(rs_sems_r if rwd else rs_sems_l).at[i % 2, s],
                device_id=(right if rwd else left,),
                device_id_type=pl.DeviceIdType.MESH,
            )

        def ag_desc(i, h, rwd):
            buf = ag_recv_r if rwd else ag_recv_l
            src = ag_own.at[i % 2] if h == 0 else buf.at[i % 2, h - 1]
            return pltpu.make_async_remote_copy(
                src_ref=src,
                dst_ref=buf.at[i % 2, h],
                send_sem=send_sems.at[i, h % 2],
                recv_sem=(ag_sems_r if rwd else ag_sems_l).at[i % 2, h],
                device_id=(right if rwd else left,),
                device_id_type=pl.DeviceIdType.MESH,
            )

        rs_sent = [[] for _ in streams]
        ag_sent = [[] for _ in streams]
        for i, (off, rwd) in enumerate(streams):
            send_buf[i, 0, :, :] = (
                trows(rs_send_chunk(0, rwd), off).astype(jnp.bfloat16)
            )
            dsc = rs_desc(i, 0, 0, rwd)
            dsc.start()
            rs_sent[i].append(dsc)

        for s in range(R_HOPS):
            for i, (off, rwd) in enumerate(streams):
                hops = n_hops[rwd]
                if s >= hops or s >= hops - 1:
                    continue
                if s >= 1:
                    rs_sent[i][s - 1].wait_send()
                send_buf[i, (s + 1) % 2, :, :] = (
                    trows(rs_send_chunk(s + 1, rwd), off).astype(jnp.bfloat16)
                )
            for i, (off, rwd) in enumerate(streams):
                hops = n_hops[rwd]
                if s >= hops:
                    continue
                recv_buf = rs_recv_r if rwd else rs_recv_l
                rs_desc(i, s, 0, rwd).wait_recv()
                if s < hops - 1:
                    slot = (s + 1) % 2
                    send_buf[i, slot, :, :] = (
                        send_buf[i, slot] + recv_buf[i % 2, s]
                    )
                    dsc = rs_desc(i, s + 1, slot, rwd)
                    dsc.start()
                    rs_sent[i].append(dsc)

        for j in range(2):
            off = j * sub
            red = (
                rs_recv_r[j, R_HOPS - 1].astype(jnp.float32)
                + rs_recv_l[j, L_HOPS - 1].astype(jnp.float32)
                + trows(d, off)
            )
            res = jnp.dot(red, w_ref[:, :], preferred_element_type=jnp.float32)
            ag_own[j, :, :] = res.astype(jnp.bfloat16)
            for i, (ioff, rwd) in enumerate(streams):
                if ioff != off:
                    continue
                rs_sent[i][n_hops[rwd] - 2].wait_send()
                rs_sent[i][n_hops[rwd] - 1].wait_send()
                dsc = ag_desc(i, 0, rwd)
                dsc.start()
                ag_sent[i].append(dsc)
            out_ref[pl.ds(d * chunk + off, sub), :] = res

        for h in range(R_HOPS):
            for i, (off, rwd) in enumerate(streams):
                hops = n_hops[rwd]
                if h >= hops:
                    continue
                if rwd:
                    rc = lax.rem(d + N_DEV - 1 - h, N_DEV)
                else:
                    rc = lax.rem(d + 1 + h, N_DEV)
                ag_desc(i, h, rwd).wait_recv()
                if h < hops - 1:
                    if h >= 1:
                        ag_sent[i][h - 1].wait_send()
                    dsc = ag_desc(i, h + 1, rwd)
                    dsc.start()
                    ag_sent[i].append(dsc)
                buf = ag_recv_r if rwd else ag_recv_l
                out_ref[pl.ds(rc * chunk + off, sub), :] = (
                    buf[i % 2, h].astype(jnp.float32)
                )
        for i, (off, rwd) in enumerate(streams):
            hops = n_hops[rwd]
            ag_sent[i][hops - 2].wait_send()
            ag_sent[i][hops - 1].wait_send()

    return pl.pallas_call(
        body,
        out_shape=jax.ShapeDtypeStruct((m, n), jnp.float32),
        in_specs=[
            pl.BlockSpec(memory_space=pltpu.VMEM),
            pl.BlockSpec(memory_space=pltpu.VMEM),
        ],
        out_specs=pl.BlockSpec(memory_space=pltpu.VMEM),
        scratch_shapes=[
            pltpu.VMEM((4, 2, sub, k), jnp.bfloat16),
            pltpu.VMEM((2, R_HOPS, sub, k), jnp.bfloat16),
            pltpu.VMEM((2, L_HOPS, sub, k), jnp.bfloat16),
            pltpu.VMEM((2, sub, k), jnp.bfloat16),
            pltpu.VMEM((2, R_HOPS, sub, k), jnp.bfloat16),
            pltpu.VMEM((2, L_HOPS, sub, k), jnp.bfloat16),
            pltpu.SemaphoreType.DMA((4, 2)),
            pltpu.SemaphoreType.DMA((2, R_HOPS)),
            pltpu.SemaphoreType.DMA((2, L_HOPS)),
            pltpu.SemaphoreType.DMA((2, R_HOPS)),
            pltpu.SemaphoreType.DMA((2, L_HOPS)),
        ],
        compiler_params=pltpu.CompilerParams(
            collective_id=0, vmem_limit_bytes=100 * 1024 * 1024
        ),
    )(t, W)


# device time: 117366 ns/iter; 3.6104x vs baseline; 1.0875x over previous
import jax
import jax.numpy as jnp
from jax import lax
from jax.experimental import pallas as pl
from jax.experimental.pallas import tpu as pltpu

N_DEV = 16
R_HOPS = 8
L_HOPS = 7


def kernel(t, W):
    m, k = t.shape
    _, n = W.shape
    chunk = m // N_DEV
    sub = chunk // 2

    def body(t_ref, w_ref, out_ref, send_buf, rs_recv_r, rs_recv_l,
             ag_own, ag_recv_r, ag_recv_l, t_stage, w_stage, out_stage,
             send_sems, rs_sems_r, rs_sems_l, ag_sems_r, ag_sems_l,
             t_sems, w_sem, out_sems):
        d = lax.axis_index("i")
        right = lax.rem(d + 1, N_DEV)
        left = lax.rem(d + N_DEV - 1, N_DEV)

        streams = [(0, True), (sub, True), (0, False), (sub, False)]
        n_hops = {True: R_HOPS, False: L_HOPS}

        barrier_sem = pltpu.get_barrier_semaphore()
        for nbr in (left, right):
            pl.semaphore_signal(
                barrier_sem, inc=1,
                device_id=(nbr,), device_id_type=pl.DeviceIdType.MESH,
            )
        pl.semaphore_wait(barrier_sem, 2)

        def rs_send_chunk(s, rwd):
            if rwd:
                return lax.rem(d + R_HOPS - s, N_DEV)
            return lax.rem(d + N_DEV - L_HOPS + s, N_DEV)

        def t_dma(i, s, rwd):
            off = streams[i][0]
            c = rs_send_chunk(s, rwd)
            return pltpu.make_async_copy(
                t_ref.at[pl.ds(c * chunk + off, sub), :],
                t_stage.at[i, s % 2],
                t_sems.at[i, s % 2],
            )

        def rs_desc(i, s, slot, rwd):
            return pltpu.make_async_remote_copy(
                src_ref=send_buf.at[i, slot],
                dst_ref=(rs_recv_r if rwd else rs_recv_l).at[i % 2, s],
                send_sem=send_sems.at[i, slot],
                recv_sem=(rs_sems_r if rwd else rs_sems_l).at[i % 2, s],
                device_id=(right if rwd else left,),
                device_id_type=pl.DeviceIdType.MESH,
            )

        def ag_desc(i, h, rwd):
            buf = ag_recv_r if rwd else ag_recv_l
            src = ag_own.at[i % 2] if h == 0 else buf.at[i % 2, h - 1]
            return pltpu.make_async_remote_copy(
                src_ref=src,
                dst_ref=buf.at[i % 2, h],
                send_sem=send_sems.at[i, h % 2],
                recv_sem=(ag_sems_r if rwd else ag_sems_l).at[i % 2, h],
                device_id=(right if rwd else left,),
                device_id_type=pl.DeviceIdType.MESH,
            )

        def out_dma(j, slot, c):
            return pltpu.make_async_copy(
                out_stage.at[j, slot],
                out_ref.at[pl.ds(c * chunk + j * sub, sub), :],
                out_sems.at[j, slot],
            )

        pltpu.make_async_copy(w_ref, w_stage, w_sem).start()
        for i, (off, rwd) in enumerate(streams):
            t_dma(i, 0, rwd).start()
            t_dma(i, 1, rwd).start()

        rs_sent = [[] for _ in streams]
        ag_sent = [[] for _ in streams]
        for i, (off, rwd) in enumerate(streams):
            t_dma(i, 0, rwd).wait()
            send_buf[i, 0, :, :] = t_stage[i, 0].astype(jnp.bfloat16)
            dsc = rs_desc(i, 0, 0, rwd)
            dsc.start()
            rs_sent[i].append(dsc)

        for s in range(R_HOPS):
            for i, (off, rwd) in enumerate(streams):
                hops = n_hops[rwd]
                if s >= hops - 1:
                    continue
                if s >= 1:
                    rs_sent[i][s - 1].wait_send()
                t_dma(i, s + 1, rwd).wait()
                send_buf[i, (s + 1) % 2, :, :] = (
                    t_stage[i, (s + 1) % 2].astype(jnp.bfloat16)
                )
                if s + 2 <= (hops if rwd else hops - 1):
                    t_dma(i, s + 2, rwd).start()
            for i, (off, rwd) in enumerate(streams):
                hops = n_hops[rwd]
                if s >= hops:
                    continue
                recv_buf = rs_recv_r if rwd else rs_recv_l
                rs_desc(i, s, 0, rwd).wait_recv()
                if s < hops - 1:
                    slot = (s + 1) % 2
                    send_buf[i, slot, :, :] = (
                        send_buf[i, slot] + recv_buf[i % 2, s]
                    )
                    dsc = rs_desc(i, s + 1, slot, rwd)
                    dsc.start()
                    rs_sent[i].append(dsc)

        pltpu.make_async_copy(w_ref, w_stage, w_sem).wait()
        for j in range(2):
            off = j * sub
            t_dma(j, R_HOPS, True).wait()
            red = (
                rs_recv_r[j, R_HOPS - 1].astype(jnp.float32)
                + rs_recv_l[j, L_HOPS - 1].astype(jnp.float32)
                + t_stage[j, R_HOPS % 2]
            )
            res = jnp.dot(
                red, w_stage[:, :], preferred_element_type=jnp.float32
            )
            ag_own[j, :, :] = res.astype(jnp.bfloat16)
            for i, (ioff, rwd) in enumerate(streams):
                if ioff != off:
                    continue
                rs_sent[i][n_hops[rwd] - 2].wait_send()
                rs_sent[i][n_hops[rwd] - 1].wait_send()
                dsc = ag_desc(i, 0, rwd)
                dsc.start()
                ag_sent[i].append(dsc)
            out_stage[j, 0, :, :] = res
            out_dma(j, 0, d).start()

        for h in range(R_HOPS):
            for i, (off, rwd) in enumerate(streams):
                hops = n_hops[rwd]
                if h >= hops:
                    continue
                j = i % 2
                if rwd:
                    rc = lax.rem(d + N_DEV - 1 - h, N_DEV)
                    slot = 1 + h
                else:
                    rc = lax.rem(d + 1 + h, N_DEV)
                    slot = 1 + R_HOPS + h
                ag_desc(i, h, rwd).wait_recv()
                if h < hops - 1:
                    if h >= 1:
                        ag_sent[i][h - 1].wait_send()
                    dsc = ag_desc(i, h + 1, rwd)
                    dsc.start()
                    ag_sent[i].append(dsc)
                buf = ag_recv_r if rwd else ag_recv_l
                out_stage[j, slot, :, :] = buf[j, h].astype(jnp.float32)
                out_dma(j, slot, rc).start()
        for i, (off, rwd) in enumerate(streams):
            hops = n_hops[rwd]
            ag_sent[i][hops - 2].wait_send()
            ag_sent[i][hops - 1].wait_send()
        for j in range(2):
            for slot in range(N_DEV):
                c = d if slot == 0 else (
                    lax.rem(d + N_DEV - slot, N_DEV) if slot <= R_HOPS
                    else lax.rem(d + slot - R_HOPS, N_DEV)
                )
                out_dma(j, slot, c).wait()

    return pl.pallas_call(
        body,
        out_shape=jax.ShapeDtypeStruct((m, n), jnp.float32),
        in_specs=[
            pl.BlockSpec(memory_space=pl.ANY),
            pl.BlockSpec(memory_space=pl.ANY),
        ],
        out_specs=pl.BlockSpec(memory_space=pl.ANY),
        scratch_shapes=[
            pltpu.VMEM((4, 2, sub, k), jnp.bfloat16),
            pltpu.VMEM((2, R_HOPS, sub, k), jnp.bfloat16),
            pltpu.VMEM((2, L_HOPS, sub, k), jnp.bfloat16),
            pltpu.VMEM((2, sub, k), jnp.bfloat16),
            pltpu.VMEM((2, R_HOPS, sub, k), jnp.bfloat16),
            pltpu.VMEM((2, L_HOPS, sub, k), jnp.bfloat16),
            pltpu.VMEM((4, 2, sub, k), jnp.float32),
            pltpu.VMEM((k, n), jnp.float32),
            pltpu.VMEM((2, N_DEV, sub, k), jnp.float32),
            pltpu.SemaphoreType.DMA((4, 2)),
            pltpu.SemaphoreType.DMA((2, R_HOPS)),
            pltpu.SemaphoreType.DMA((2, L_HOPS)),
            pltpu.SemaphoreType.DMA((2, R_HOPS)),
            pltpu.SemaphoreType.DMA((2, L_HOPS)),
            pltpu.SemaphoreType.DMA((4, 2)),
            pltpu.SemaphoreType.DMA(()),
            pltpu.SemaphoreType.DMA((2, N_DEV)),
        ],
        compiler_params=pltpu.CompilerParams(
            collective_id=0, vmem_limit_bytes=100 * 1024 * 1024
        ),
    )(t, W)


# device time: 117243 ns/iter; 3.6142x vs baseline; 1.0010x over previous
import jax
import jax.numpy as jnp
from jax import lax
from jax.experimental import pallas as pl
from jax.experimental.pallas import tpu as pltpu

N_DEV = 16
R_HOPS = 8
L_HOPS = 7


def kernel(t, W):
    m, k = t.shape
    _, n = W.shape
    chunk = m // N_DEV
    sub = chunk // 4

    def body(t_ref, w_ref, out_ref, send_buf, rs_recv_r, rs_recv_l,
             ag_own, ag_recv_r, ag_recv_l, t_stage, w_stage, out_stage,
             send_sems, rs_sems_r, rs_sems_l, ag_sems_r, ag_sems_l,
             t_sems, w_sem, out_sems):
        d = lax.axis_index("i")
        right = lax.rem(d + 1, N_DEV)
        left = lax.rem(d + N_DEV - 1, N_DEV)

        streams = [(j * sub, True) for j in range(4)] + [
            (j * sub, False) for j in range(4)
        ]
        n_hops = {True: R_HOPS, False: L_HOPS}

        barrier_sem = pltpu.get_barrier_semaphore()
        for nbr in (left, right):
            pl.semaphore_signal(
                barrier_sem, inc=1,
                device_id=(nbr,), device_id_type=pl.DeviceIdType.MESH,
            )
        pl.semaphore_wait(barrier_sem, 2)

        def rs_send_chunk(s, rwd):
            if rwd:
                return lax.rem(d + R_HOPS - s, N_DEV)
            return lax.rem(d + N_DEV - L_HOPS + s, N_DEV)

        def t_dma(i, s, rwd):
            off = streams[i][0]
            c = rs_send_chunk(s, rwd)
            return pltpu.make_async_copy(
                t_ref.at[pl.ds(c * chunk + off, sub), :],
                t_stage.at[i, s % 2],
                t_sems.at[i, s % 2],
            )

        def rs_desc(i, s, slot, rwd):
            return pltpu.make_async_remote_copy(
                src_ref=send_buf.at[i, slot],
                dst_ref=(rs_recv_r if rwd else rs_recv_l).at[i % 4, s],
                send_sem=send_sems.at[i, slot],
                recv_sem=(rs_sems_r if rwd else rs_sems_l).at[i % 4, s],
                device_id=(right if rwd else left,),
                device_id_type=pl.DeviceIdType.MESH,
            )

        def ag_desc(i, h, rwd):
            buf = ag_recv_r if rwd else ag_recv_l
            src = ag_own.at[i % 4] if h == 0 else buf.at[i % 4, h - 1]
            return pltpu.make_async_remote_copy(
                src_ref=src,
                dst_ref=buf.at[i % 4, h],
                send_sem=send_sems.at[i, h % 2],
                recv_sem=(ag_sems_r if rwd else ag_sems_l).at[i % 4, h],
                device_id=(right if rwd else left,),
                device_id_type=pl.DeviceIdType.MESH,
            )

        def out_dma(j, slot, c):
            return pltpu.make_async_copy(
                out_stage.at[j, slot],
                out_ref.at[pl.ds(c * chunk + j * sub, sub), :],
                out_sems.at[j, slot],
            )

        pltpu.make_async_copy(w_ref, w_stage, w_sem).start()
        for i, (off, rwd) in enumerate(streams):
            t_dma(i, 0, rwd).start()
            t_dma(i, 1, rwd).start()

        rs_sent = [[] for _ in streams]
        ag_sent = [[] for _ in streams]
        for i, (off, rwd) in enumerate(streams):
            t_dma(i, 0, rwd).wait()
            send_buf[i, 0, :, :] = t_stage[i, 0].astype(jnp.bfloat16)
            dsc = rs_desc(i, 0, 0, rwd)
            dsc.start()
            rs_sent[i].append(dsc)

        for s in range(R_HOPS):
            for i, (off, rwd) in enumerate(streams):
                hops = n_hops[rwd]
                if s >= hops - 1:
                    continue
                if s >= 1:
                    rs_sent[i][s - 1].wait_send()
                t_dma(i, s + 1, rwd).wait()
                send_buf[i, (s + 1) % 2, :, :] = (
                    t_stage[i, (s + 1) % 2].astype(jnp.bfloat16)
                )
                if s + 2 <= (hops if rwd else hops - 1):
                    t_dma(i, s + 2, rwd).start()
            for i, (off, rwd) in enumerate(streams):
                hops = n_hops[rwd]
                if s >= hops:
                    continue
                recv_buf = rs_recv_r if rwd else rs_recv_l
                rs_desc(i, s, 0, rwd).wait_recv()
                if s < hops - 1:
                    slot = (s + 1) % 2
                    send_buf[i, slot, :, :] = (
                        send_buf[i, slot] + recv_buf[i % 4, s]
                    )
                    dsc = rs_desc(i, s + 1, slot, rwd)
                    dsc.start()
                    rs_sent[i].append(dsc)

        pltpu.make_async_copy(w_ref, w_stage, w_sem).wait()
        for j in range(4):
            off = j * sub
            t_dma(j, R_HOPS, True).wait()
            red = (
                rs_recv_r[j, R_HOPS - 1].astype(jnp.float32)
                + rs_recv_l[j, L_HOPS - 1].astype(jnp.float32)
                + t_stage[j, R_HOPS % 2]
            )
            res = jnp.dot(
                red, w_stage[:, :], preferred_element_type=jnp.float32
            )
            ag_own[j, :, :] = res.astype(jnp.bfloat16)
            for i, (ioff, rwd) in enumerate(streams):
                if ioff != off:
                    continue
                rs_sent[i][n_hops[rwd] - 2].wait_send()
                rs_sent[i][n_hops[rwd] - 1].wait_send()
                dsc = ag_desc(i, 0, rwd)
                dsc.start()
                ag_sent[i].append(dsc)
            out_stage[j, 0, :, :] = res
            out_dma(j, 0, d).start()

        for h in range(R_HOPS):
            for i, (off, rwd) in enumerate(streams):
                hops = n_hops[rwd]
                if h >= hops:
                    continue
                j = i % 4
                if rwd:
                    rc = lax.rem(d + N_DEV - 1 - h, N_DEV)
                    slot = 1 + h
                else:
                    rc = lax.rem(d + 1 + h, N_DEV)
                    slot = 1 + R_HOPS + h
                ag_desc(i, h, rwd).wait_recv()
                if h < hops - 1:
                    if h >= 1:
                        ag_sent[i][h - 1].wait_send()
                    dsc = ag_desc(i, h + 1, rwd)
                    dsc.start()
                    ag_sent[i].append(dsc)
                buf = ag_recv_r if rwd else ag_recv_l
                out_stage[j, slot, :, :] = buf[j, h].astype(jnp.float32)
                out_dma(j, slot, rc).start()
        for i, (off, rwd) in enumerate(streams):
            hops = n_hops[rwd]
            ag_sent[i][hops - 2].wait_send()
            ag_sent[i][hops - 1].wait_send()
        for j in range(4):
            for slot in range(N_DEV):
                c = d if slot == 0 else (
                    lax.rem(d + N_DEV - slot, N_DEV) if slot <= R_HOPS
                    else lax.rem(d + slot - R_HOPS, N_DEV)
                )
                out_dma(j, slot, c).wait()

    return pl.pallas_call(
        body,
        out_shape=jax.ShapeDtypeStruct((m, n), jnp.float32),
        in_specs=[
            pl.BlockSpec(memory_space=pl.ANY),
            pl.BlockSpec(memory_space=pl.ANY),
        ],
        out_specs=pl.BlockSpec(memory_space=pl.ANY),
        scratch_shapes=[
            pltpu.VMEM((8, 2, sub, k), jnp.bfloat16),
            pltpu.VMEM((4, R_HOPS, sub, k), jnp.bfloat16),
            pltpu.VMEM((4, L_HOPS, sub, k), jnp.bfloat16),
            pltpu.VMEM((4, sub, k), jnp.bfloat16),
            pltpu.VMEM((4, R_HOPS, sub, k), jnp.bfloat16),
            pltpu.VMEM((4, L_HOPS, sub, k), jnp.bfloat16),
            pltpu.VMEM((8, 2, sub, k), jnp.float32),
            pltpu.VMEM((k, n), jnp.float32),
            pltpu.VMEM((4, N_DEV, sub, k), jnp.float32),
            pltpu.SemaphoreType.DMA((8, 2)),
            pltpu.SemaphoreType.DMA((4, R_HOPS)),
            pltpu.SemaphoreType.DMA((4, L_HOPS)),
            pltpu.SemaphoreType.DMA((4, R_HOPS)),
            pltpu.SemaphoreType.DMA((4, L_HOPS)),
            pltpu.SemaphoreType.DMA((8, 2)),
            pltpu.SemaphoreType.DMA(()),
            pltpu.SemaphoreType.DMA((4, N_DEV)),
        ],
        compiler_params=pltpu.CompilerParams(
            collective_id=0, vmem_limit_bytes=100 * 1024 * 1024
        ),
    )(t, W)
